# baseline (speedup 1.0000x reference)
import functools

import jax
import jax.numpy as jnp
from jax import lax
from jax.experimental import pallas as pl
from jax.experimental.pallas import tpu as pltpu
from jax.experimental.pallas import tpu_sc as plsc

EPSILON = 0.1
D = 128
NCH = D // 16
L = 10
NS = 3
CTX_IDX = 8
NWORK = 32
GROWS = L + 2
SWORDS = L + 1
SROWS = NS * SWORDS


def _sc_body(wg_hbm, ws_hbm, idxg_hbm, idxs_hbm, x2s_hbm, out_hbm,
             idxg_v, idxs_v, x2s_v, wg_buf, ws_buf, out_v,
             sg0, sg1, sg2, sg3, ss0, ss1, ss2, ss3, epw):
    wid = lax.axis_index("s") * 2 + lax.axis_index("c")
    semg = (sg0, sg1, sg2, sg3)
    sems = (ss0, ss1, ss2, ss3)

    pltpu.sync_copy(idxg_hbm.at[wid], idxg_v)
    pltpu.sync_copy(idxs_hbm.at[wid], idxs_v)
    pltpu.sync_copy(x2s_hbm.at[wid], x2s_v)

    def gather_descs(e, b):
        g = pltpu.make_async_copy(wg_hbm.at[idxg_v.at[e]], wg_buf.at[b], semg[b])
        s = pltpu.make_async_copy(ws_hbm.at[idxs_v.at[e]], ws_buf.at[b], sems[b])
        return g, s

    def start(e, b):
        g, s = gather_descs(e, b)
        g.start()
        s.start()

    iota16 = lax.iota(jnp.int32, 16)

    def lane_sum(v):
        for sh in (8, 4, 2, 1):
            v = v + v.at[iota16 ^ sh].get(mode="promise_in_bounds")
        return v

    def compute(e, b):
        g, s = gather_descs(e, b)
        g.wait()
        s.wait()
        wg = wg_buf.at[b]
        ws = ws_buf.at[b]

        cv = []
        for dc in range(NCH):
            a = wg[0, pl.ds(dc * 16, 16)]
            for i in range(1, L):
                a = a + wg[i, pl.ds(dc * 16, 16)]
            cv.append(a)

        x2v = x2s_v[e, :]
        sxv = jnp.zeros((16,), jnp.int32)
        s8 = jnp.zeros((16,), jnp.int32)
        one16 = jnp.ones((16,), jnp.int32)
        zero16 = jnp.zeros((16,), jnp.int32)
        two16 = jnp.full((16,), 2, jnp.int32)

        for i in range(L):
            w = [[ws[SWORDS * sn + i, pl.ds(dc * 16, 16)] for dc in range(NCH)]
                 for sn in range(NS)]
            sc = []
            for sn in range(NS):
                a = w[sn][0] * cv[0]
                for dc in range(1, NCH):
                    a = a + w[sn][dc] * cv[dc]
                sc.append(lane_sum(a))
            s0, s1, s2 = sc
            m01 = jnp.maximum(s0, s1)
            lo01 = jnp.minimum(s0, s1)
            i01 = jnp.where(s1 >= s0, one16, zero16)
            best = jnp.where(s2 >= m01, two16, i01)
            best_sc = jnp.maximum(m01, s2)
            second_sc = jnp.where(s2 >= m01, m01, jnp.maximum(lo01, s2))
            upd = (best_sc - second_sc) > EPSILON
            for dc in range(NCH):
                wb = jnp.where(best == 0, w[0][dc],
                               jnp.where(best == 1, w[1][dc], w[2][dc]))
                cv[dc] = jnp.where(upd,
                                   cv[dc] - wg[i, pl.ds(dc * 16, 16)] + wb,
                                   cv[dc])
            if i == CTX_IDX:
                s8 = best
            sxv = jnp.where(x2v == i, best, sxv)

        accs = jnp.zeros((16,), jnp.float32)
        accg = jnp.zeros((16,), jnp.float32)
        for dc in range(NCH):
            r0 = ws[SWORDS * 0 + L, pl.ds(dc * 16, 16)]
            r1 = ws[SWORDS * 1 + L, pl.ds(dc * 16, 16)]
            r2 = ws[SWORDS * 2 + L, pl.ds(dc * 16, 16)]
            wsv = jnp.where(s8 == 0, r0, jnp.where(s8 == 1, r1, r2))
            csv = jnp.where(sxv == 0, r0, jnp.where(sxv == 1, r1, r2))
            accs = accs + wsv * csv
            accg = accg + wg[L, pl.ds(dc * 16, 16)] * wg[L + 1, pl.ds(dc * 16, 16)]
        logit = lane_sum(accs) + lane_sum(accg)
        r = 1.0 / (1.0 + jnp.exp(-logit))
        out_v[e, :] = r

    NB = 4
    for b in range(NB):
        start(b, b)

    def loop_body(gi, carry):
        for b in range(NB):
            e = NB * gi + b
            compute(e, b)

            @pl.when(gi < epw // NB - 1)
            def _():
                start(e + NB, b)
        return carry

    lax.fori_loop(0, epw // NB, loop_body, jnp.int32(0))
    pltpu.sync_copy(out_v, out_hbm.at[wid])


def kernel(x, W_g, W_s):
    B = x.shape[0]
    vocab = W_g.shape[0]
    epw = B // NWORK

    x = x.astype(jnp.int32)
    x0 = x[:, 0:1]
    x1 = x[:, 1:2]
    x2 = x[:, 2]
    ctx = x[:, 3:3 + L]
    idxg = jnp.concatenate([ctx, x0, x1], axis=1).reshape(NWORK, epw, GROWS)
    words = jnp.concatenate([ctx, x0], axis=1)
    ws_flat = jnp.transpose(W_s, (1, 0, 2)).reshape(NS * vocab, D)
    idxs = (jnp.arange(NS, dtype=jnp.int32)[None, :, None] * vocab
            + words[:, None, :]).reshape(NWORK, epw, SROWS)
    x2s = jnp.broadcast_to(x2[:, None], (B, 16)).reshape(NWORK, epw, 16)

    grid_kernel = pl.kernel(
        functools.partial(_sc_body, epw=epw),
        out_type=jax.ShapeDtypeStruct((NWORK, epw, 16), jnp.float32),
        mesh=plsc.VectorSubcoreMesh(core_axis_name="c", subcore_axis_name="s"),
        scratch_types=[
            pltpu.VMEM((epw, GROWS), jnp.int32),
            pltpu.VMEM((epw, SROWS), jnp.int32),
            pltpu.VMEM((epw, 16), jnp.int32),
            pltpu.VMEM((4, GROWS, D), jnp.float32),
            pltpu.VMEM((4, SROWS, D), jnp.float32),
            pltpu.VMEM((epw, 16), jnp.float32),
            pltpu.SemaphoreType.DMA,
            pltpu.SemaphoreType.DMA,
            pltpu.SemaphoreType.DMA,
            pltpu.SemaphoreType.DMA,
            pltpu.SemaphoreType.DMA,
            pltpu.SemaphoreType.DMA,
            pltpu.SemaphoreType.DMA,
            pltpu.SemaphoreType.DMA,
        ],
    )
    out = grid_kernel(W_g, ws_flat, idxg, idxs, x2s)
    return out[:, :, 0].reshape(B, 1)


# speedup vs baseline: 1.7682x; 1.7682x over previous
import functools

import jax
import jax.numpy as jnp
from jax import lax
from jax.experimental import pallas as pl
from jax.experimental.pallas import tpu as pltpu
from jax.experimental.pallas import tpu_sc as plsc

EPSILON = 0.1
D = 128
NCH = D // 16
L = 10
NS = 3
CTX_IDX = 8
NWORK = 32
GROWS = L + 2
SWORDS = L + 1
SROWS = NS * SWORDS
CEX = 3


def _sc_body(wg_hbm, ws_hbm, idxg_hbm, idxs_hbm, x2s_hbm, out_hbm,
             idxg_v, idxs_v, x2s_v, wg_buf, ws_buf, out_v,
             sg0, sg1, ss0, ss1, nchk):
    wid = lax.axis_index("s") * 2 + lax.axis_index("c")
    semg = (sg0, sg1)
    sems = (ss0, ss1)

    pltpu.sync_copy(idxg_hbm.at[wid], idxg_v)
    pltpu.sync_copy(idxs_hbm.at[wid], idxs_v)
    pltpu.sync_copy(x2s_hbm.at[wid], x2s_v)

    def gather_descs(c, b):
        g = pltpu.make_async_copy(wg_hbm.at[idxg_v.at[c]], wg_buf.at[b], semg[b])
        s = pltpu.make_async_copy(ws_hbm.at[idxs_v.at[c]], ws_buf.at[b], sems[b])
        return g, s

    def start(c, b):
        g, s = gather_descs(c, b)
        g.start()
        s.start()

    iota16 = lax.iota(jnp.int32, 16)

    def lane_sum(v):
        for sh in (8, 4, 2, 1):
            v = v + v.at[iota16 ^ sh].get(mode="promise_in_bounds")
        return v

    def compute(e, jg, js, b):
        wg = wg_buf.at[b]
        ws = ws_buf.at[b]

        cv = []
        for dc in range(NCH):
            a = wg[jg, pl.ds(dc * 16, 16)]
            for i in range(1, L):
                a = a + wg[jg + i, pl.ds(dc * 16, 16)]
            cv.append(a)

        x2v = x2s_v[e, :]
        sxv = jnp.zeros((16,), jnp.int32)
        s8 = jnp.zeros((16,), jnp.int32)
        one16 = jnp.ones((16,), jnp.int32)
        zero16 = jnp.zeros((16,), jnp.int32)
        two16 = jnp.full((16,), 2, jnp.int32)

        for i in range(L):
            w = [[ws[js + SWORDS * sn + i, pl.ds(dc * 16, 16)] for dc in range(NCH)]
                 for sn in range(NS)]
            sc = []
            for sn in range(NS):
                a = w[sn][0] * cv[0]
                for dc in range(1, NCH):
                    a = a + w[sn][dc] * cv[dc]
                sc.append(lane_sum(a))
            s0, s1, s2 = sc
            m01 = jnp.maximum(s0, s1)
            lo01 = jnp.minimum(s0, s1)
            i01 = jnp.where(s1 >= s0, one16, zero16)
            best = jnp.where(s2 >= m01, two16, i01)
            best_sc = jnp.maximum(m01, s2)
            second_sc = jnp.where(s2 >= m01, m01, jnp.maximum(lo01, s2))
            upd = (best_sc - second_sc) > EPSILON
            for dc in range(NCH):
                wb = jnp.where(best == 0, w[0][dc],
                               jnp.where(best == 1, w[1][dc], w[2][dc]))
                cv[dc] = jnp.where(upd,
                                   cv[dc] - wg[jg + i, pl.ds(dc * 16, 16)] + wb,
                                   cv[dc])
            if i == CTX_IDX:
                s8 = best
            sxv = jnp.where(x2v == i, best, sxv)

        accs = jnp.zeros((16,), jnp.float32)
        accg = jnp.zeros((16,), jnp.float32)
        for dc in range(NCH):
            r0 = ws[js + SWORDS * 0 + L, pl.ds(dc * 16, 16)]
            r1 = ws[js + SWORDS * 1 + L, pl.ds(dc * 16, 16)]
            r2 = ws[js + SWORDS * 2 + L, pl.ds(dc * 16, 16)]
            wsv = jnp.where(s8 == 0, r0, jnp.where(s8 == 1, r1, r2))
            csv = jnp.where(sxv == 0, r0, jnp.where(sxv == 1, r1, r2))
            accs = accs + wsv * csv
            accg = accg + wg[jg + L, pl.ds(dc * 16, 16)] * wg[jg + L + 1, pl.ds(dc * 16, 16)]
        logit = lane_sum(accs) + lane_sum(accg)
        r = 1.0 / (1.0 + jnp.exp(-logit))
        out_v[e, :] = r

    for b in range(2):
        start(b, b)

    def loop_body(gi, carry):
        for b in range(2):
            c = 2 * gi + b
            g, s = gather_descs(c, b)
            g.wait()
            s.wait()

            def inner(j, cc):
                compute(CEX * c + j, j * GROWS, j * SROWS, b)
                return cc

            lax.fori_loop(0, CEX, inner, jnp.int32(0))

            @pl.when(gi < nchk // 2 - 1)
            def _():
                start(c + 2, b)
        return carry

    lax.fori_loop(0, nchk // 2, loop_body, jnp.int32(0))
    pltpu.sync_copy(out_v, out_hbm.at[wid])


def kernel(x, W_g, W_s):
    B = x.shape[0]
    vocab = W_g.shape[0]
    epw = B // NWORK
    nchk = -(-epw // CEX)
    epad = nchk * CEX

    x = x.astype(jnp.int32)
    x0 = x[:, 0:1]
    x1 = x[:, 1:2]
    x2 = x[:, 2]
    ctx = x[:, 3:3 + L]
    idxg = jnp.concatenate([ctx, x0, x1], axis=1).reshape(NWORK, epw, GROWS)
    words = jnp.concatenate([ctx, x0], axis=1)
    ws_flat = jnp.transpose(W_s, (1, 0, 2)).reshape(NS * vocab, D)
    idxs = (jnp.arange(NS, dtype=jnp.int32)[None, :, None] * vocab
            + words[:, None, :]).reshape(NWORK, epw, SROWS)
    x2s = jnp.broadcast_to(x2[:, None], (B, 16)).reshape(NWORK, epw, 16)

    npad = epad - epw
    if npad:
        idxg = jnp.concatenate([idxg, idxg[:, -npad:, :]], axis=1)
        idxs = jnp.concatenate([idxs, idxs[:, -npad:, :]], axis=1)
        x2s = jnp.concatenate([x2s, x2s[:, -npad:, :]], axis=1)
    idxg = idxg.reshape(NWORK, nchk, CEX * GROWS)
    idxs = idxs.reshape(NWORK, nchk, CEX * SROWS)

    grid_kernel = pl.kernel(
        functools.partial(_sc_body, nchk=nchk),
        out_type=jax.ShapeDtypeStruct((NWORK, epad, 16), jnp.float32),
        mesh=plsc.VectorSubcoreMesh(core_axis_name="c", subcore_axis_name="s"),
        scratch_types=[
            pltpu.VMEM((nchk, CEX * GROWS), jnp.int32),
            pltpu.VMEM((nchk, CEX * SROWS), jnp.int32),
            pltpu.VMEM((epad, 16), jnp.int32),
            pltpu.VMEM((2, CEX * GROWS, D), jnp.float32),
            pltpu.VMEM((2, CEX * SROWS, D), jnp.float32),
            pltpu.VMEM((epad, 16), jnp.float32),
            pltpu.SemaphoreType.DMA,
            pltpu.SemaphoreType.DMA,
            pltpu.SemaphoreType.DMA,
            pltpu.SemaphoreType.DMA,
        ],
    )
    out = grid_kernel(W_g, ws_flat, idxg, idxs, x2s)
    return out[:, :epw, 0].reshape(B, 1)
